# Initial kernel scaffold; baseline (speedup 1.0000x reference)
import jax
import jax.numpy as jnp
from jax.experimental import pallas as pl


def kernel(state_wf, state_vm, edge_index_wf, edge_index_vm, mask_wf, mask_vm, batch_wf, batch_vm, candidate_task_index, w0_1, b0_1, g0, be0, w0_2, b0_2, w1_1, b1_1, g1, be1, w1_2, b1_2, wa1, ba1, wa2, ba2):
    raise NotImplementedError("write your pallas kernel here")


# speedup vs baseline: 8.0744x; 8.0744x over previous
import functools

import jax
import jax.numpy as jnp
from jax import lax
from jax.experimental import pallas as pl
from jax.experimental.pallas import tpu as pltpu
from jax.experimental.pallas import tpu_sc as plsc

NC, NS = 2, 16
NW = NC * NS
CHUNK = 128
H = 128


def _sc_mesh():
    return plsc.VectorSubcoreMesh(
        core_axis_name="c", subcore_axis_name="s", num_cores=NC, num_subcores=NS
    )


def _make_sc_agg(n, n_alloc, n_chunks):
    HW = H // 2
    rows_per_tile = n // NS
    zsizes = [CHUNK] * (rows_per_tile // CHUNK)
    if rows_per_tile % CHUNK:
        zsizes.append(rows_per_tile % CHUNK)

    def body(xs, src, dst, zb, part, isrc, idst, rows, agg_sh, sem):
        c = lax.axis_index("c")
        s = lax.axis_index("s")
        base = s * rows_per_tile

        pltpu.sync_copy(src.at[c, s], isrc)
        pltpu.sync_copy(dst.at[s], idst)
        pltpu.sync_copy(zb, rows)
        off = 0
        for zs in zsizes:
            pltpu.sync_copy(rows.at[pl.ds(0, zs)],
                            agg_sh.at[pl.ds(base + off, zs)])
            off += zs
        plsc.subcore_barrier()

        def step(j, carry):
            pltpu.async_copy(xs.at[isrc.at[j]], rows, sem).wait()
            pltpu.sync_copy(rows, agg_sh.at[idst.at[j]], add=True)
            return carry

        lax.fori_loop(0, n_chunks, step, 0)
        plsc.subcore_barrier()

        pltpu.sync_copy(agg_sh.at[pl.ds(base, rows_per_tile)],
                        part.at[c].at[pl.ds(base, rows_per_tile)])

    return pl.kernel(
        body,
        out_type=jax.ShapeDtypeStruct((NC, n, HW), jnp.float32),
        mesh=_sc_mesh(),
        scratch_types=[
            pltpu.VMEM((n_chunks, CHUNK), jnp.int32),
            pltpu.VMEM((n_chunks, CHUNK), jnp.int32),
            pltpu.VMEM((CHUNK, HW), jnp.float32),
            pltpu.VMEM_SHARED((n_alloc, HW), jnp.float32),
            pltpu.SemaphoreType.DMA,
        ],
        compiler_params=pltpu.CompilerParams(use_tc_tiling_on_sc=False),
    )


def _make_sc_deg(n, n_alloc, n_chunks):
    rows_per_tile = n // NS
    zsizes = [CHUNK] * (rows_per_tile // CHUNK)
    if rows_per_tile % CHUNK:
        zsizes.append(rows_per_tile % CHUNK)

    def body(dst, z16, o16, degp, idst, buf16, deg_sh):
        c = lax.axis_index("c")
        s = lax.axis_index("s")
        wid = c * NS + s
        base = s * rows_per_tile

        pltpu.sync_copy(dst.at[wid], idst)
        pltpu.sync_copy(z16, buf16)
        off = 0
        for zs in zsizes:
            pltpu.sync_copy(buf16.at[pl.ds(0, zs)],
                            deg_sh.at[pl.ds(base + off, zs)])
            off += zs
        pltpu.sync_copy(o16, buf16)
        plsc.subcore_barrier()

        def step(j, carry):
            pltpu.sync_copy(buf16, deg_sh.at[idst.at[j]], add=True)
            return carry

        lax.fori_loop(0, n_chunks, step, 0)
        plsc.subcore_barrier()

        pltpu.sync_copy(deg_sh.at[pl.ds(base, rows_per_tile)],
                        degp.at[c].at[pl.ds(base, rows_per_tile)])

    return pl.kernel(
        body,
        out_type=jax.ShapeDtypeStruct((NC, n, 16), jnp.float32),
        mesh=_sc_mesh(),
        scratch_types=[
            pltpu.VMEM((n_chunks, CHUNK), jnp.int32),
            pltpu.VMEM((CHUNK, 16), jnp.float32),
            pltpu.VMEM_SHARED((n_alloc, 16), jnp.float32),
        ],
        compiler_params=pltpu.CompilerParams(use_tc_tiling_on_sc=False),
    )


def _make_sc_gather(n_rows_out, chunks_per_tile):
    per_tile = chunks_per_tile * CHUNK

    def body(x, idx, out, iv, rows, sem):
        c = lax.axis_index("c")
        s = lax.axis_index("s")
        wid = c * NS + s
        pltpu.sync_copy(idx.at[wid], iv)
        for k in range(chunks_per_tile):
            pltpu.async_copy(x.at[iv.at[k]], rows, sem).wait()
            pltpu.sync_copy(rows, out.at[pl.ds(wid * per_tile + k * CHUNK, CHUNK)])

    return pl.kernel(
        body,
        out_type=jax.ShapeDtypeStruct((n_rows_out, H), jnp.float32),
        mesh=_sc_mesh(),
        scratch_types=[
            pltpu.VMEM((chunks_per_tile, CHUNK), jnp.int32),
            pltpu.VMEM((CHUNK, H), jnp.float32),
            pltpu.SemaphoreType.DMA,
        ],
    )


_BLK = 1000


def _dense1_body(x, p0, p1, d0, d1, w, b, y):
    deg = d0[:, 0:1] + d1[:, 0:1]
    agg = jnp.concatenate([p0[...], p1[...]], axis=1)
    z = x[...] + agg / jnp.maximum(deg, 1.0)
    y[...] = jnp.dot(z, w[...], preferred_element_type=jnp.float32) + b[...]


def _dense1(x, p0, p1, d0, d1, w, b, n_rows):
    ngrid = n_rows // _BLK
    return pl.pallas_call(
        _dense1_body,
        grid=(ngrid,),
        in_specs=[
            pl.BlockSpec((_BLK, H), lambda i: (i, 0)),
            pl.BlockSpec((_BLK, H // 2), lambda i: (i, 0)),
            pl.BlockSpec((_BLK, H // 2), lambda i: (i, 0)),
            pl.BlockSpec((_BLK, 16), lambda i: (i, 0)),
            pl.BlockSpec((_BLK, 16), lambda i: (i, 0)),
            pl.BlockSpec((H, H), lambda i: (0, 0)),
            pl.BlockSpec((1, H), lambda i: (0, 0)),
        ],
        out_specs=pl.BlockSpec((_BLK, H), lambda i: (i, 0)),
        out_shape=jax.ShapeDtypeStruct((n_rows, H), jnp.float32),
    )(x, p0, p1, d0, d1, w, b)


def _dense2_body(relu_out, y, st, g, be, w2, b2, o):
    mean = st[0:1, :]
    var = st[1:2, :]
    inv = 1.0 / jnp.sqrt(var + 1e-5)
    h = (y[...] - mean) * inv * g[...] + be[...]
    h = jnp.maximum(h, 0.0)
    ov = jnp.dot(h, w2[...], preferred_element_type=jnp.float32) + b2[...]
    if relu_out:
        ov = jnp.maximum(ov, 0.0)
    o[...] = ov


def _dense2(y, st, g, be, w2, b2, n_rows, relu_out):
    ngrid = n_rows // _BLK
    body = functools.partial(_dense2_body, relu_out)
    return pl.pallas_call(
        body,
        grid=(ngrid,),
        in_specs=[
            pl.BlockSpec((_BLK, H), lambda i: (i, 0)),
            pl.BlockSpec((8, H), lambda i: (0, 0)),
            pl.BlockSpec((1, H), lambda i: (0, 0)),
            pl.BlockSpec((1, H), lambda i: (0, 0)),
            pl.BlockSpec((H, H), lambda i: (0, 0)),
            pl.BlockSpec((1, H), lambda i: (0, 0)),
        ],
        out_specs=pl.BlockSpec((_BLK, H), lambda i: (i, 0)),
        out_shape=jax.ShapeDtypeStruct((n_rows, H), jnp.float32),
    )(y, st, g, be, w2, b2)


def _mlp_body(cand, wa1, ba1, wa2p, ba2, s):
    hc = jnp.dot(cand[...], wa1[...], preferred_element_type=jnp.float32) + ba1[...]
    hc = jnp.maximum(hc, 0.0)
    sv = jnp.dot(hc, wa2p[...], preferred_element_type=jnp.float32) + ba2[...]
    s[...] = sv[:, 0:1]


def _mlp(cand, wa1, ba1, wa2p, ba2, n_rows):
    full = lambda shp: pl.BlockSpec(shp, lambda: (0,) * len(shp))
    return pl.pallas_call(
        _mlp_body,
        in_specs=[
            full((n_rows, H)),
            full((H, H)),
            full((1, H)),
            full((H, H)),
            full((1, 1)),
        ],
        out_specs=full((n_rows, 1)),
        out_shape=jax.ShapeDtypeStruct((n_rows, 1), jnp.float32),
    )(cand, wa1, ba1, wa2p, ba2)


def _head_body(nb, s_ref, a_ref, lp_ref, ent_ref):
    s = s_ref[...]
    m = jnp.max(s, axis=1, keepdims=True)
    e = jnp.exp(s - m)
    den = jnp.sum(e, axis=1, keepdims=True)
    pi = e / den
    lane = lax.broadcasted_iota(jnp.int32, s.shape, 1)
    act = jnp.min(jnp.where(s >= m, lane, s.shape[1]), axis=1, keepdims=True)
    a_ref[...] = jnp.broadcast_to(act, s.shape)
    pa = jnp.sum(jnp.where(lane == act, pi, 0.0), axis=1, keepdims=True)
    lp_ref[...] = jnp.broadcast_to(jnp.log(pa + 1e-20), s.shape)
    ent = -jnp.sum(pi * jnp.log(pi + 1e-20)) / nb
    ent_ref[...] = jnp.broadcast_to(ent, ent_ref.shape)


def _head(scores_pad, nb):
    full = lambda shp: pl.BlockSpec(shp, lambda: (0,) * len(shp))
    return pl.pallas_call(
        functools.partial(_head_body, float(nb)),
        in_specs=[full((nb, H))],
        out_specs=[full((nb, H)), full((nb, H)), full((8, H))],
        out_shape=[
            jax.ShapeDtypeStruct((nb, H), jnp.int32),
            jax.ShapeDtypeStruct((nb, H), jnp.float32),
            jax.ShapeDtypeStruct((8, H), jnp.float32),
        ],
    )(scores_pad)


def kernel(state_wf, state_vm, edge_index_wf, edge_index_vm, mask_wf, mask_vm,
           batch_wf, batch_vm, candidate_task_index,
           w0_1, b0_1, g0, be0, w0_2, b0_2,
           w1_1, b1_1, g1, be1, w1_2, b1_2,
           wa1, ba1, wa2, ba2):
    n = state_wf.shape[0]
    e2 = 2 * edge_index_wf.shape[1]
    n_alloc = ((n + 64) // 64) * 64
    ep = ((e2 + NW * CHUNK - 1) // (NW * CHUNK)) * NW * CHUNK
    n_chunks = ep // (NS * CHUNK)
    n_chunks_w = ep // (NW * CHUNK)

    src = jnp.concatenate([edge_index_wf[0], edge_index_wf[1],
                           jnp.zeros((ep - e2,), jnp.int32)])
    dst = jnp.concatenate([edge_index_wf[1], edge_index_wf[0],
                           jnp.full((ep - e2,), n, jnp.int32)])
    src3 = src.reshape(NS, n_chunks, CHUNK)
    src4 = jnp.stack([src3, src3 + n])
    dst3 = dst.reshape(NS, n_chunks, CHUNK)
    dstw = dst.reshape(NW, n_chunks_w, CHUNK)

    zb = jnp.zeros((CHUNK, H // 2), jnp.float32)
    z16 = jnp.zeros((CHUNK, 16), jnp.float32)
    o16 = jnp.ones((CHUNK, 16), jnp.float32)

    def split(v):
        return jnp.concatenate([v[:, :H // 2], v[:, H // 2:]], axis=0)

    b_ = lambda v: v.reshape(1, -1)

    degp = _make_sc_deg(n, n_alloc, n_chunks_w)(dstw, z16, o16)
    agg_fn = _make_sc_agg(n, n_alloc, n_chunks)
    part = agg_fn(split(state_wf), src4, dst3, zb)

    def stats(y):
        mean = jnp.mean(y, axis=0)
        var = jnp.var(y, axis=0)
        return jnp.concatenate([mean.reshape(1, H), var.reshape(1, H),
                                jnp.zeros((6, H), jnp.float32)], axis=0)

    y0 = _dense1(state_wf, part[0], part[1], degp[0], degp[1],
                 w0_1, b_(b0_1), n)
    h1 = _dense2(y0, stats(y0), b_(g0), b_(be0), w0_2, b_(b0_2), n, True)

    part2 = agg_fn(split(h1), src4, dst3, zb)
    y1 = _dense1(h1, part2[0], part2[1], degp[0], degp[1],
                 w1_1, b_(b1_1), n)
    h2 = _dense2(y1, stats(y1), b_(g1), b_(be1), w1_2, b_(b1_2), n, False)

    ncand = candidate_task_index.shape[0]
    cpt = ncand // (NW * CHUNK)
    idx3 = candidate_task_index.reshape(NW, cpt, CHUNK)
    cand = _make_sc_gather(ncand, cpt)(h2, idx3)

    wa2p = jnp.pad(wa2.reshape(H, 1), ((0, 0), (0, H - 1)))
    scores = _mlp(cand, wa1, b_(ba1), wa2p, ba2.reshape(1, 1), ncand)

    vm = 16
    nb = ncand // vm
    s2 = scores.reshape(nb, vm)
    s_pad = jnp.pad(s2, ((0, 0), (0, H - vm)), constant_values=-1e30)
    a2, lp2, ent2 = _head(s_pad, nb)

    actions = a2[:, 0]
    logp = lp2[:, 0]
    ent = ent2[0, 0]
    return (actions, logp, ent)
